# Initial kernel scaffold; baseline (speedup 1.0000x reference)
import jax
import jax.numpy as jnp
from jax.experimental import pallas as pl


def kernel(equipment, randomness_source, equipment_states):
    raise NotImplementedError("write your pallas kernel here")


# speedup vs baseline: 1.0102x; 1.0102x over previous
import jax
import jax.numpy as jnp
from jax.experimental import pallas as pl

REPAIR_P = 0.3
DEGRADE_P = 0.1
CRITICAL_P = 0.01

_ROWS_PER_BLOCK = 256


def _body(eq_ref, rnd_ref, out_ref, *, S):
    eq = eq_ref[...]
    rnd = rnd_ref[...]
    damaged = eq == 0
    pristine = eq == (S - 1)
    r_rep = rnd < REPAIR_P
    r_deg = rnd < DEGRADE_P
    r_crit = rnd < CRITICAL_P
    rep_val = jnp.where(r_rep, jnp.int32(S - 1), jnp.int32(0))
    crit = jnp.logical_and(pristine, r_crit)
    nd_val = jnp.where(crit, jnp.int32(0), jnp.where(r_deg, eq - 1, eq))
    out_ref[...] = jnp.where(damaged, rep_val, nd_val)


def kernel(equipment, randomness_source, equipment_states):
    S = equipment_states.shape[0]
    R, C = equipment.shape
    grid = (R // _ROWS_PER_BLOCK,)
    spec = pl.BlockSpec((_ROWS_PER_BLOCK, C), lambda i: (i, 0))
    import functools
    return pl.pallas_call(
        functools.partial(_body, S=S),
        grid=grid,
        in_specs=[spec, spec],
        out_specs=spec,
        out_shape=jax.ShapeDtypeStruct((R, C), jnp.int32),
    )(equipment, randomness_source)
